# Initial kernel scaffold; baseline (speedup 1.0000x reference)
import jax
import jax.numpy as jnp
from jax.experimental import pallas as pl


def kernel(idx, targets, table):
    raise NotImplementedError("write your pallas kernel here")


# speedup vs baseline: 1.3907x; 1.3907x over previous
import functools

import jax
import jax.numpy as jnp
from jax import lax
from jax.experimental import pallas as pl
from jax.experimental.pallas import tpu as pltpu
from jax.experimental.pallas import tpu_sc as plsc

V = 1000
N = 1024 * 50
NC, NS, L = 2, 16, 16
NW = NC * NS
ROWS_PER_W = N // NW
CHUNK = 80
NCHUNK = ROWS_PER_W // CHUNK


def _lse_body(table_ref, out_ref):
    x = table_ref[...]
    m = jnp.max(x, axis=1, keepdims=True)
    out_ref[...] = (jnp.log(jnp.sum(jnp.exp(x - m), axis=1, keepdims=True))
                    + m).reshape(1, V)


def _gather_body(table_hbm, idx_hbm, tgt_hbm, lse_hbm, out_hbm, part_hbm,
                 idx_v, tgt_v, rows_v, lse_v, acc_v, gsem):
    wid = lax.axis_index("s") * NC + lax.axis_index("c")
    base = wid * ROWS_PER_W

    pltpu.sync_copy(lse_hbm, lse_v)
    acc_v[...] = jnp.zeros((L,), jnp.float32)

    def chunk_step(k, carry):
        off = base + k * CHUNK
        pltpu.sync_copy(idx_hbm.at[pl.ds(off, CHUNK)], idx_v)
        pltpu.sync_copy(tgt_hbm.at[pl.ds(off, CHUNK)], tgt_v)
        pltpu.async_copy(table_hbm.at[idx_v], rows_v, gsem).wait()
        pltpu.sync_copy(rows_v, out_hbm.at[pl.ds(off, CHUNK)])

        def loss_step(j, c):
            iv = idx_v[pl.ds(j * L, L)]
            tv = tgt_v[pl.ds(j * L, L)]
            rid = j * L + lax.broadcasted_iota(jnp.int32, (L,), 0)
            lse_vals = plsc.load_gather(lse_v, [iv])
            tgt_vals = plsc.load_gather(rows_v, [rid, tv])
            acc_v[...] = acc_v[...] + (lse_vals - tgt_vals)
            return c

        return lax.fori_loop(0, CHUNK // L, loss_step, carry)

    lax.fori_loop(0, NCHUNK, chunk_step, 0)
    pltpu.sync_copy(acc_v, part_hbm.at[wid])


def _finish_body(part_ref, out_ref):
    out_ref[...] = jnp.sum(part_ref[...] * (1.0 / N), keepdims=True).reshape(1, 1)


@jax.jit
def kernel(idx, targets, table):
    lse = pl.pallas_call(
        _lse_body,
        out_shape=jax.ShapeDtypeStruct((1, V), jnp.float32),
    )(table)

    mesh = plsc.VectorSubcoreMesh(core_axis_name="c", subcore_axis_name="s")
    gather = functools.partial(
        pl.kernel,
        out_type=[
            jax.ShapeDtypeStruct((N, V), jnp.float32),
            jax.ShapeDtypeStruct((NW, L), jnp.float32),
        ],
        mesh=mesh,
        compiler_params=pltpu.CompilerParams(
            needs_layout_passes=False, use_tc_tiling_on_sc=False),
        scratch_types=[
            pltpu.VMEM((CHUNK,), jnp.int32),
            pltpu.VMEM((CHUNK,), jnp.int32),
            pltpu.VMEM((CHUNK, V), jnp.float32),
            pltpu.VMEM((V,), jnp.float32),
            pltpu.VMEM((L,), jnp.float32),
            pltpu.SemaphoreType.DMA,
        ],
    )(_gather_body)
    logits_flat, partials = gather(
        table, idx.reshape(N), targets.reshape(N), lse.reshape(V))

    loss = pl.pallas_call(
        _finish_body,
        out_shape=jax.ShapeDtypeStruct((1, 1), jnp.float32),
    )(partials)

    return logits_flat.reshape(idx.shape + (V,)), loss.reshape(())


# speedup vs baseline: 1.4347x; 1.0316x over previous
import functools

import jax
import jax.numpy as jnp
from jax import lax
from jax.experimental import pallas as pl
from jax.experimental.pallas import tpu as pltpu
from jax.experimental.pallas import tpu_sc as plsc

V = 1000
B, T = 1024, 50
N = B * T
TP = 56
NC, NS, L = 2, 16, 16
NW = NC * NS
B_PER_W = B // NW
PAD_PER_W = B_PER_W * TP


def _lse_body(table_ref, out_ref):
    x = table_ref[...]
    m = jnp.max(x, axis=1, keepdims=True)
    out_ref[...] = (jnp.log(jnp.sum(jnp.exp(x - m), axis=1, keepdims=True))
                    + m).reshape(1, V)


def _gather_body(table_hbm, idx_hbm, tgt_hbm, lse_hbm, out_hbm, part_hbm,
                 idx_v, tgt_v, rows0_v, rows1_v, lse_v, acc_v,
                 gsem0, gsem1, osem0, osem1):
    wid = lax.axis_index("s") * NC + lax.axis_index("c")
    base = wid * PAD_PER_W
    b0 = wid * B_PER_W

    pltpu.sync_copy(lse_hbm, lse_v)
    pltpu.sync_copy(idx_hbm.at[pl.ds(base, PAD_PER_W)],
                    idx_v.at[pl.ds(0, PAD_PER_W)])
    pltpu.sync_copy(tgt_hbm.at[pl.ds(base, PAD_PER_W)],
                    tgt_v.at[pl.ds(0, PAD_PER_W)])
    acc_v[...] = jnp.zeros((L,), jnp.float32)

    iota = lax.broadcasted_iota(jnp.int32, (L,), 0)

    def loss_row(j, rows_buf):
        for i in range(4):
            off = j * TP + i * L
            pos = i * L + iota
            m = pos < T
            iv = jnp.clip(idx_v[pl.ds(off, L)], 0, V - 1)
            tv = jnp.clip(tgt_v[pl.ds(off, L)], 0, V - 1)
            rid = jnp.minimum(pos, T - 1)
            lse_vals = plsc.load_gather(lse_v, [iv])
            tlg = plsc.load_gather(rows_buf, [rid, tv])
            acc_v[...] = acc_v[...] + jnp.where(m, lse_vals - tlg, 0.0)

    bufs = (rows0_v, rows1_v)
    gsems = (gsem0, gsem1)
    osems = (osem0, osem1)

    def launch_gather(j):
        return pltpu.async_copy(
            table_hbm.at[idx_v.at[pl.ds(j * TP, T)]], bufs[j % 2],
            gsems[j % 2])

    g_pend = [launch_gather(0), None]
    o_pend = [None, None]
    for j in range(B_PER_W):
        p = j % 2
        q = (j + 1) % 2
        if j + 1 < B_PER_W:
            if o_pend[q] is not None:
                o_pend[q].wait()
            g_pend[q] = launch_gather(j + 1)
        g_pend[p].wait()
        o_pend[p] = pltpu.async_copy(bufs[p], out_hbm.at[b0 + j], osems[p])
        loss_row(j, bufs[p])
    o_pend[0].wait()
    o_pend[1].wait()

    pltpu.sync_copy(acc_v, part_hbm.at[wid])


def _finish_body(part_ref, out_ref):
    out_ref[...] = jnp.sum(part_ref[...] * (1.0 / N), keepdims=True).reshape(1, 1)


@jax.jit
def kernel(idx, targets, table):
    lse = pl.pallas_call(
        _lse_body,
        out_shape=jax.ShapeDtypeStruct((1, V), jnp.float32),
    )(table)

    idx_p = jnp.pad(idx, ((0, 0), (0, TP - T))).reshape(B * TP)
    tgt_p = jnp.pad(targets, ((0, 0), (0, TP - T))).reshape(B * TP)

    mesh = plsc.VectorSubcoreMesh(core_axis_name="c", subcore_axis_name="s")
    gather = functools.partial(
        pl.kernel,
        out_type=[
            jax.ShapeDtypeStruct((B, T, V), jnp.float32),
            jax.ShapeDtypeStruct((NW, L), jnp.float32),
        ],
        mesh=mesh,
        compiler_params=pltpu.CompilerParams(
            needs_layout_passes=False, use_tc_tiling_on_sc=False),
        scratch_types=[
            pltpu.VMEM((PAD_PER_W + L,), jnp.int32),
            pltpu.VMEM((PAD_PER_W + L,), jnp.int32),
            pltpu.VMEM((T, V), jnp.float32),
            pltpu.VMEM((T, V), jnp.float32),
            pltpu.VMEM((V,), jnp.float32),
            pltpu.VMEM((L,), jnp.float32),
            pltpu.SemaphoreType.DMA,
            pltpu.SemaphoreType.DMA,
            pltpu.SemaphoreType.DMA,
            pltpu.SemaphoreType.DMA,
        ],
    )(_gather_body)
    logits, partials = gather(table, idx_p, tgt_p, lse.reshape(V))

    loss = pl.pallas_call(
        _finish_body,
        out_shape=jax.ShapeDtypeStruct((1, 1), jnp.float32),
    )(partials)

    return logits, loss.reshape(())


# speedup vs baseline: 2.3782x; 1.6577x over previous
import functools

import jax
import jax.numpy as jnp
from jax import lax
from jax.experimental import pallas as pl
from jax.experimental.pallas import tpu as pltpu
from jax.experimental.pallas import tpu_sc as plsc

V = 1000
VMAIN = 896
VTAIL = V - VMAIN
CB = VMAIN // 128
B, T = 1024, 50
N = B * T
TP = 56
NC, NS, L = 2, 16, 16
NW = NC * NS
B_PER_W = B // NW
PAD_PER_W = B_PER_W * TP
TAIL_OFFS = (0, 16, 32, 48, 64, 80, 88)


def _lse_body(table_ref, out_ref):
    x = table_ref[...]
    m = jnp.max(x, axis=1, keepdims=True)
    out_ref[...] = (jnp.log(jnp.sum(jnp.exp(x - m), axis=1, keepdims=True))
                    + m).reshape(1, V)


def _gather_body(table7_hbm, tlast_hbm, idx_hbm, tgt_hbm, lse_hbm,
                 out_hbm, part_hbm, *refs):
    blk = (refs[0:8], refs[8:16])
    tail_v, idx_v, tgt_v, lse_v, acc_v = refs[16:21]
    gsems = refs[21:23]
    osems = refs[23:25]
    tsem = refs[25]

    wid = lax.axis_index("s") * NC + lax.axis_index("c")
    base = wid * PAD_PER_W
    b0 = wid * B_PER_W

    pltpu.sync_copy(lse_hbm, lse_v)
    pltpu.sync_copy(idx_hbm.at[pl.ds(base, PAD_PER_W)],
                    idx_v.at[pl.ds(0, PAD_PER_W)])
    pltpu.sync_copy(tgt_hbm.at[pl.ds(base, PAD_PER_W)],
                    tgt_v.at[pl.ds(0, PAD_PER_W)])
    acc_v[...] = jnp.zeros((L,), jnp.float32)

    iota = lax.broadcasted_iota(jnp.int32, (L,), 0)

    def loss_row(j, bufs):
        for i in range(4):
            off = j * TP + i * L
            pos = i * L + iota
            m = pos < T
            iv = jnp.clip(idx_v[pl.ds(off, L)], 0, V - 1)
            tv = jnp.clip(tgt_v[pl.ds(off, L)], 0, V - 1)
            rid = jnp.minimum(pos, T - 1)
            cbv = lax.shift_right_logical(tv, 7)
            cin = lax.bitwise_and(tv, 127)
            lse_vals = plsc.load_gather(lse_v, [iv])
            tlg = plsc.load_gather(
                bufs[CB], [rid, jnp.clip(tv - VMAIN, 0, 127)])
            for k in range(CB):
                gk = plsc.load_gather(bufs[k], [rid, cin])
                tlg = jnp.where(cbv == k, gk, tlg)
            acc_v[...] = acc_v[...] + jnp.where(m, lse_vals - tlg, 0.0)

    def repack_tail(last_buf):
        def body(r, c):
            for o in TAIL_OFFS:
                tail_v[r, pl.ds(o, L)] = last_buf[r, pl.ds(o, L)]
            return c
        lax.fori_loop(0, T, body, 0)

    def launch_gathers(j):
        idx_ref = idx_v.at[pl.ds(j * TP, T)]
        p = j % 2
        cps = [
            pltpu.async_copy(table7_hbm.at[k].at[idx_ref], blk[p][k],
                             gsems[p])
            for k in range(CB)
        ]
        cps.append(pltpu.async_copy(tlast_hbm.at[idx_ref], blk[p][CB],
                                    gsems[p]))
        return cps

    g_pend = [launch_gathers(0), None]
    o_pend = [None, None]
    t_pend = None
    for j in range(B_PER_W):
        p = j % 2
        q = (j + 1) % 2
        if j + 1 < B_PER_W:
            if o_pend[q] is not None:
                for o in o_pend[q]:
                    o.wait()
            g_pend[q] = launch_gathers(j + 1)
        for g in g_pend[p]:
            g.wait()
        if t_pend is not None:
            t_pend.wait()
        repack_tail(blk[p][CB])
        o_pend[p] = [
            pltpu.async_copy(blk[p][k],
                             out_hbm.at[b0 + j].at[:, pl.ds(k * 128, 128)],
                             osems[p])
            for k in range(CB)
        ]
        t_pend = pltpu.async_copy(
            tail_v, out_hbm.at[b0 + j].at[:, pl.ds(VMAIN, VTAIL)], tsem)
        loss_row(j, blk[p])
    for o in o_pend[0]:
        o.wait()
    for o in o_pend[1]:
        o.wait()
    t_pend.wait()

    pltpu.sync_copy(acc_v, part_hbm.at[pl.ds(wid * L, L)])


def _finish_body(part_ref, out_ref):
    out_ref[...] = jnp.sum(part_ref[...] * (1.0 / N), keepdims=True).reshape(1, 1)


@jax.jit
def kernel(idx, targets, table):
    lse = pl.pallas_call(
        _lse_body,
        out_shape=jax.ShapeDtypeStruct((1, V), jnp.float32),
    )(table)

    idx_p = jnp.pad(idx, ((0, 0), (0, TP - T))).reshape(B * TP)
    tgt_p = jnp.pad(targets, ((0, 0), (0, TP - T))).reshape(B * TP)
    table7 = jnp.transpose(table[:, :VMAIN].reshape(V, CB, 128), (1, 0, 2))
    tlast = jnp.pad(table[:, VMAIN:], ((0, 0), (0, 128 - VTAIL)))

    mesh = plsc.VectorSubcoreMesh(core_axis_name="c", subcore_axis_name="s")
    gather = functools.partial(
        pl.kernel,
        out_type=[
            jax.ShapeDtypeStruct((B, T, V), jnp.float32),
            jax.ShapeDtypeStruct((NW * L,), jnp.float32),
        ],
        mesh=mesh,
        compiler_params=pltpu.CompilerParams(
            needs_layout_passes=False, use_tc_tiling_on_sc=True),
        scratch_types=(
            [pltpu.VMEM((T, 128), jnp.float32)] * 16
            + [
                pltpu.VMEM((T, VTAIL), jnp.float32),
                pltpu.VMEM((PAD_PER_W + L,), jnp.int32),
                pltpu.VMEM((PAD_PER_W + L,), jnp.int32),
                pltpu.VMEM((V,), jnp.float32),
                pltpu.VMEM((L,), jnp.float32),
            ]
            + [pltpu.SemaphoreType.DMA] * 5
        ),
    )(_gather_body)
    logits, partials = gather(table7, tlast, idx_p, tgt_p, lse.reshape(V))

    loss = pl.pallas_call(
        _finish_body,
        out_shape=jax.ShapeDtypeStruct((1, 1), jnp.float32),
    )(partials.reshape(NW, L))

    return logits, loss.reshape(())
